# baseline (speedup 1.0000x reference)
import functools

import jax
import jax.numpy as jnp
from jax import lax
from jax.experimental import pallas as pl
from jax.experimental.pallas import tpu as pltpu
from jax.experimental.pallas import tpu_sc as plsc

N = 50000
E = 800000
IN = 128
HID = 64
EMB = 32
G = 64

NC = 2
NS = 16
NW = NC * NS
CH = 128
NCHUNK = 196
SB_CH = 28
SB_E = SB_CH * CH
NSB = NCHUNK // SB_CH
EPT = NCHUNK * CH
E_PAD = NW * EPT
NA = 50176
TPT = NA // NS
BLK = 3584
NBLK = NA // BLK
NAP = NA // 4
BLKP = BLK // 4

_MESH = plsc.VectorSubcoreMesh(
    core_axis_name="c", subcore_axis_name="s", num_cores=NC, num_subcores=NS)


def _sc_prop_body(table, ep, zeros, out, src_v, dst_v, rows0, rows1,
                  acc, gsem0, gsem1, ssem0, ssem1):
    c = lax.axis_index("c")
    s = lax.axis_index("s")
    wid = s * NC + c
    pltpu.sync_copy(zeros.at[pl.ds(s * TPT, TPT)], acc.at[pl.ds(s * TPT, TPT)])
    plsc.subcore_barrier()

    @pl.loop(0, NSB)
    def _(b):
        base = wid * EPT + b * SB_E
        pltpu.sync_copy(ep.at[0, pl.ds(base, SB_E)], src_v)
        pltpu.sync_copy(ep.at[1, pl.ds(base, SB_E)], dst_v)
        pltpu.async_copy(table.at[src_v.at[pl.ds(0, CH)]], rows0, gsem0)
        pltpu.async_copy(table.at[src_v.at[pl.ds(CH, CH)]], rows1, gsem1)

        @pl.loop(0, SB_CH, step=2)
        def _(j):
            pltpu.make_async_copy(
                table.at[src_v.at[pl.ds(j * CH, CH)]], rows0, gsem0).wait()
            pltpu.async_copy(rows0, acc.at[dst_v.at[pl.ds(j * CH, CH)]],
                             ssem0, add=True)
            pltpu.make_async_copy(
                table.at[src_v.at[pl.ds((j + 1) * CH, CH)]], rows1,
                gsem1).wait()
            pltpu.async_copy(rows1, acc.at[dst_v.at[pl.ds((j + 1) * CH, CH)]],
                             ssem1, add=True)

            @pl.when(j + 2 < SB_CH)
            def _():
                pltpu.make_async_copy(
                    rows0, acc.at[dst_v.at[pl.ds(j * CH, CH)]], ssem0).wait()
                pltpu.async_copy(
                    table.at[src_v.at[pl.ds((j + 2) * CH, CH)]], rows0, gsem0)

            @pl.when(j + 3 < SB_CH)
            def _():
                pltpu.make_async_copy(
                    rows1, acc.at[dst_v.at[pl.ds((j + 1) * CH, CH)]],
                    ssem1).wait()
                pltpu.async_copy(
                    table.at[src_v.at[pl.ds((j + 3) * CH, CH)]], rows1, gsem1)

        pltpu.make_async_copy(
            rows0, acc.at[dst_v.at[pl.ds((SB_CH - 2) * CH, CH)]],
            ssem0).wait()
        pltpu.make_async_copy(
            rows1, acc.at[dst_v.at[pl.ds((SB_CH - 1) * CH, CH)]],
            ssem1).wait()

    plsc.subcore_barrier()
    pltpu.sync_copy(acc.at[pl.ds(s * TPT, TPT)], out.at[c, pl.ds(s * TPT, TPT)])


_sc_prop = pl.kernel(
    _sc_prop_body,
    out_type=jax.ShapeDtypeStruct((NC, NA, EMB), jnp.float32),
    mesh=_MESH,
    compiler_params=pltpu.CompilerParams(use_tc_tiling_on_sc=False),
    scratch_types=[
        pltpu.VMEM((SB_E,), jnp.int32),
        pltpu.VMEM((SB_E,), jnp.int32),
        pltpu.VMEM((CH, EMB), jnp.float32),
        pltpu.VMEM((CH, EMB), jnp.float32),
        pltpu.VMEM_SHARED((NA, EMB), jnp.float32),
        pltpu.SemaphoreType.DMA,
        pltpu.SemaphoreType.DMA,
        pltpu.SemaphoreType.DMA,
        pltpu.SemaphoreType.DMA,
    ],
)


def _sc_deg_body(ep, ones, zeros8, out, dst_v, ones_v, acc, sem):
    c = lax.axis_index("c")
    s = lax.axis_index("s")
    wid = s * NC + c
    pltpu.sync_copy(zeros8.at[pl.ds(s * TPT, TPT)], acc.at[pl.ds(s * TPT, TPT)])
    pltpu.sync_copy(ones, ones_v)
    plsc.subcore_barrier()

    @pl.loop(0, NSB)
    def _(b):
        pltpu.sync_copy(ep.at[1, pl.ds(wid * EPT + b * SB_E, SB_E)], dst_v)

        @pl.loop(0, SB_CH)
        def _(j):
            pltpu.async_copy(ones_v, acc.at[dst_v.at[pl.ds(j * CH, CH)]],
                             sem, add=True)

        @pl.loop(0, SB_CH)
        def _(j):
            pltpu.make_async_copy(
                ones_v, acc.at[dst_v.at[pl.ds(j * CH, CH)]], sem).wait()

    plsc.subcore_barrier()
    pltpu.sync_copy(acc.at[pl.ds(s * TPT, TPT)], out.at[c, pl.ds(s * TPT, TPT)])


_sc_deg = pl.kernel(
    _sc_deg_body,
    out_type=jax.ShapeDtypeStruct((NC, NA, 8), jnp.float32),
    mesh=_MESH,
    compiler_params=pltpu.CompilerParams(use_tc_tiling_on_sc=False),
    scratch_types=[
        pltpu.VMEM((SB_E,), jnp.int32),
        pltpu.VMEM((CH, 8), jnp.float32),
        pltpu.VMEM_SHARED((NA, 8), jnp.float32),
        pltpu.SemaphoreType.DMA,
    ],
)


def _mm(a, b):
    return lax.dot_general(a, b, (((1,), (0,)), ((), ())),
                           preferred_element_type=jnp.float32)


def _stage0_body(xp_ref, wep_ref, bep_ref, degp_ref, seld_ref, t1_ref,
                 dinv_ref):
    i = pl.program_id(0)
    h0 = _mm(xp_ref[...], wep_ref[...]) + bep_ref[...]
    deg8 = degp_ref[0] + degp_ref[1]
    deg = _mm(deg8, seld_ref[...]) + 1.0
    rows = lax.broadcasted_iota(jnp.int32, (BLKP, 128), 0)
    lanes = lax.broadcasted_iota(jnp.int32, (BLKP, 128), 1)
    node = 4 * (i * BLKP + rows) + lanes // EMB
    dinv = jnp.where(node < N, lax.rsqrt(deg), 0.0)
    t1_ref[...] = dinv * h0
    dinv_ref[...] = dinv


def _tc_stage0(xp, WeP, beP, degp4, SelD):
    return pl.pallas_call(
        _stage0_body,
        grid=(NBLK,),
        in_specs=[
            pl.BlockSpec((BLKP, 4 * IN), lambda i: (i, 0)),
            pl.BlockSpec((4 * IN, 128), lambda i: (0, 0)),
            pl.BlockSpec((1, 128), lambda i: (0, 0)),
            pl.BlockSpec((NC, BLKP, EMB), lambda i: (0, i, 0)),
            pl.BlockSpec((EMB, 128), lambda i: (0, 0)),
        ],
        out_specs=[
            pl.BlockSpec((BLKP, 128), lambda i: (i, 0)),
            pl.BlockSpec((BLKP, 128), lambda i: (i, 0)),
        ],
        out_shape=[
            jax.ShapeDtypeStruct((NAP, 128), jnp.float32),
            jax.ShapeDtypeStruct((NAP, 128), jnp.float32),
        ],
    )(xp, WeP, beP, degp4, SelD)


def _layer1_body(p_ref, t1_ref, dinv_ref, w1_ref, b1_ref, sela_ref, selb_ref,
                 a_ref, b_ref):
    dinv = dinv_ref[...]
    pre = dinv * (p_ref[0] + p_ref[1] + t1_ref[...])
    h1 = jax.nn.relu(_mm(pre, w1_ref[...]) + b1_ref[...])
    a_ref[...] = dinv * _mm(h1, sela_ref[...])
    b_ref[...] = dinv * _mm(h1, selb_ref[...])


def _tc_layer1(P14, t1p, dinvp, W1P, b1P, SelA, SelB):
    return pl.pallas_call(
        _layer1_body,
        grid=(NBLK,),
        in_specs=[
            pl.BlockSpec((NC, BLKP, 128), lambda i: (0, i, 0)),
            pl.BlockSpec((BLKP, 128), lambda i: (i, 0)),
            pl.BlockSpec((BLKP, 128), lambda i: (i, 0)),
            pl.BlockSpec((128, 256), lambda i: (0, 0)),
            pl.BlockSpec((1, 256), lambda i: (0, 0)),
            pl.BlockSpec((256, 128), lambda i: (0, 0)),
            pl.BlockSpec((256, 128), lambda i: (0, 0)),
        ],
        out_specs=[
            pl.BlockSpec((BLKP, 128), lambda i: (i, 0)),
            pl.BlockSpec((BLKP, 128), lambda i: (i, 0)),
        ],
        out_shape=[
            jax.ShapeDtypeStruct((NAP, 128), jnp.float32),
            jax.ShapeDtypeStruct((NAP, 128), jnp.float32),
        ],
    )(P14, t1p, dinvp, W1P, b1P, SelA, SelB)


def _layer2_body(pa_ref, pb_ref, ta_ref, tb_ref, dinv_ref, w2a_ref, w2b_ref,
                 b2_ref, w3_ref, t3_ref):
    dinv = dinv_ref[...]
    col_a = dinv * (pa_ref[0] + pa_ref[1] + ta_ref[...])
    col_b = dinv * (pb_ref[0] + pb_ref[1] + tb_ref[...])
    h2 = jax.nn.relu(_mm(col_a, w2a_ref[...]) + _mm(col_b, w2b_ref[...])
                     + b2_ref[...])
    t3_ref[...] = dinv * _mm(h2, w3_ref[...])


def _tc_layer2(P2a4, P2b4, t2ap, t2bp, dinvp, W2aP, W2bP, b2P, W3P):
    return pl.pallas_call(
        _layer2_body,
        grid=(NBLK,),
        in_specs=[
            pl.BlockSpec((NC, BLKP, 128), lambda i: (0, i, 0)),
            pl.BlockSpec((NC, BLKP, 128), lambda i: (0, i, 0)),
            pl.BlockSpec((BLKP, 128), lambda i: (i, 0)),
            pl.BlockSpec((BLKP, 128), lambda i: (i, 0)),
            pl.BlockSpec((BLKP, 128), lambda i: (i, 0)),
            pl.BlockSpec((128, 256), lambda i: (0, 0)),
            pl.BlockSpec((128, 256), lambda i: (0, 0)),
            pl.BlockSpec((1, 256), lambda i: (0, 0)),
            pl.BlockSpec((256, 128), lambda i: (0, 0)),
        ],
        out_specs=[pl.BlockSpec((BLKP, 128), lambda i: (i, 0))],
        out_shape=[jax.ShapeDtypeStruct((NAP, 128), jnp.float32)],
    )(P2a4, P2b4, t2ap, t2bp, dinvp, W2aP, W2bP, b2P, W3P)[0]


def _pool_body(p_ref, t3_ref, dinv_ref, b3_ref, batch_ref, s_ref, c_ref):
    i = pl.program_id(0)
    out3 = dinv_ref[...] * (p_ref[0] + p_ref[1] + t3_ref[...]) + b3_ref[...]
    iota_g = lax.broadcasted_iota(jnp.int32, (G, BLKP), 0)
    ones_b = jnp.ones((BLKP, EMB), jnp.float32)
    s_part = jnp.zeros((G, EMB), jnp.float32)
    c_part = jnp.zeros((G, EMB), jnp.float32)
    for q in range(4):
        oh = jnp.where(batch_ref[q:q + 1, :] == iota_g, 1.0, 0.0)
        s_part += _mm(oh, out3[:, EMB * q:EMB * (q + 1)])
        c_part += _mm(oh, ones_b)

    @pl.when(i == 0)
    def _():
        s_ref[...] = jnp.zeros_like(s_ref)
        c_ref[...] = jnp.zeros_like(c_ref)

    s_ref[...] += s_part
    c_ref[...] += c_part


def _tc_pool(P34, t3p, dinvp, b3P, batchp4):
    return pl.pallas_call(
        _pool_body,
        grid=(NBLK,),
        in_specs=[
            pl.BlockSpec((NC, BLKP, 128), lambda i: (0, i, 0)),
            pl.BlockSpec((BLKP, 128), lambda i: (i, 0)),
            pl.BlockSpec((BLKP, 128), lambda i: (i, 0)),
            pl.BlockSpec((1, 128), lambda i: (0, 0)),
            pl.BlockSpec((8, BLKP), lambda i: (0, i)),
        ],
        out_specs=[
            pl.BlockSpec((G, EMB), lambda i: (0, 0)),
            pl.BlockSpec((G, EMB), lambda i: (0, 0)),
        ],
        out_shape=[
            jax.ShapeDtypeStruct((G, EMB), jnp.float32),
            jax.ShapeDtypeStruct((G, EMB), jnp.float32),
        ],
    )(P34, t3p, dinvp, b3P, batchp4)


def _dotT(a, w):
    return lax.dot_general(a, w, (((1,), (1,)), ((), ())),
                           preferred_element_type=jnp.float32)


def _heads_body(s_ref, c_ref, wv_ref, bv_ref, wo_ref, bo_ref,
                tw1_ref, tb1_ref, tw2_ref, tb2_ref, tw3_ref, tb3_ref,
                iw1_ref, ib1_ref, iw2_ref, ib2_ref, iw3_ref, ib3_ref,
                aw1_ref, ab1_ref, aw2_ref, ab2_ref,
                temp_ref, iaw_ref, anom_ref, hf_ref):
    mean = s_ref[...] / jnp.maximum(c_ref[...], 1.0)
    v = _dotT(mean, wv_ref[...]) + bv_ref[...]
    hf = _dotT(v, wo_ref[...]) + bo_ref[...]
    hf_ref[...] = hf
    t = jax.nn.relu(_dotT(hf, tw1_ref[...]) + tb1_ref[...])
    t = jax.nn.relu(_dotT(t, tw2_ref[...]) + tb2_ref[...])
    temp_ref[...] = jax.nn.sigmoid(_dotT(t, tw3_ref[...]) + tb3_ref[0, 0])[:, 0:1]
    w = jax.nn.relu(_dotT(hf, iw1_ref[...]) + ib1_ref[...])
    w = jax.nn.relu(_dotT(w, iw2_ref[...]) + ib2_ref[...])
    iaw_ref[...] = jax.nn.relu(_dotT(w, iw3_ref[...]) + ib3_ref[0, 0])[:, 0:1]
    a = jax.nn.relu(_dotT(hf, aw1_ref[...]) + ab1_ref[...])
    lg = _dotT(a, aw2_ref[...]) + ab2_ref[...]
    l0, l1 = lg[:, 0:1], lg[:, 1:2]
    m = jnp.maximum(l0, l1)
    e0, e1 = jnp.exp(l0 - m), jnp.exp(l1 - m)
    tot = e0 + e1
    anom_ref[...] = jnp.concatenate([e0 / tot, e1 / tot], axis=1)


def _tc_heads(s, c, Wv, bv2, Wo, bo2, Tw1, Tb12, Tw2, Tb22, Tw3, Tb32,
              Iw1, Ib12, Iw2, Ib22, Iw3, Ib32, Aw1, Ab12, Aw2, Ab22):
    return pl.pallas_call(
        _heads_body,
        out_shape=[
            jax.ShapeDtypeStruct((G, 1), jnp.float32),
            jax.ShapeDtypeStruct((G, 1), jnp.float32),
            jax.ShapeDtypeStruct((G, 2), jnp.float32),
            jax.ShapeDtypeStruct((G, EMB), jnp.float32),
        ],
    )(s, c, Wv, bv2, Wo, bo2, Tw1, Tb12, Tw2, Tb22, Tw3, Tb32,
      Iw1, Ib12, Iw2, Ib22, Iw3, Ib32, Aw1, Ab12, Aw2, Ab22)


def kernel(x, edge_index, batch, We, be, W1, b1, W2, b2, W3, b3, Wqkv, bqkv,
           Wo, bo, Tw1, Tb1, Tw2, Tb2, Tw3, Tb3, Iw1, Ib1, Iw2, Ib2, Iw3, Ib3,
           Aw1, Ab1, Aw2, Ab2):
    f32 = jnp.float32
    eye4 = jnp.eye(4, dtype=f32)
    kron = jnp.kron
    ep = jnp.pad(edge_index, ((0, 0), (0, E_PAD - E)), constant_values=N)
    batch_pad = jnp.concatenate([batch, jnp.full((NA - N,), G, jnp.int32)])
    batchp4 = jnp.concatenate(
        [batch_pad.reshape(NAP, 4).T, jnp.full((4, NAP), G, jnp.int32)], axis=0)
    zeros32 = jnp.zeros((NA, EMB), f32)
    zeros8 = jnp.zeros((NA, 8), f32)
    ones8 = jnp.ones((CH, 8), f32)

    xp = jnp.concatenate(
        [x.reshape(N // 4, 4 * IN),
         jnp.zeros((NAP - N // 4, 4 * IN), f32)])
    WeP = kron(eye4, We.T)
    beP = jnp.tile(be, 4).reshape(1, 128)
    SelD = kron(eye4, jnp.full((8, EMB), 0.125, f32))
    W1P = kron(eye4, W1.T)
    b1P = jnp.tile(b1, 4).reshape(1, 256)
    SelA = kron(eye4, jnp.eye(HID, EMB, dtype=f32))
    SelB = kron(eye4, jnp.eye(HID, EMB, k=-EMB, dtype=f32))
    W2aP = kron(eye4, W2[:, :EMB].T)
    W2bP = kron(eye4, W2[:, EMB:].T)
    b2P = jnp.tile(b2, 4).reshape(1, 256)
    W3P = kron(eye4, W3.T)
    b3P = jnp.tile(b3, 4).reshape(1, 128)

    degp = _sc_deg(ep, ones8, zeros8)
    t1p, dinvp = _tc_stage0(xp, WeP, beP, degp.reshape(NC, NAP, EMB), SelD)
    P1 = _sc_prop(t1p.reshape(NA, EMB), ep, zeros32)
    t2ap, t2bp = _tc_layer1(P1.reshape(NC, NAP, 128), t1p, dinvp, W1P, b1P,
                            SelA, SelB)
    P2a = _sc_prop(t2ap.reshape(NA, EMB), ep, zeros32)
    P2b = _sc_prop(t2bp.reshape(NA, EMB), ep, zeros32)
    t3p = _tc_layer2(P2a.reshape(NC, NAP, 128), P2b.reshape(NC, NAP, 128),
                     t2ap, t2bp, dinvp, W2aP, W2bP, b2P, W3P)
    P3 = _sc_prop(t3p.reshape(NA, EMB), ep, zeros32)
    s, c = _tc_pool(P3.reshape(NC, NAP, 128), t3p, dinvp, b3P, batchp4)
    pad8 = lambda w: jnp.concatenate(
        [w, jnp.zeros((8 - w.shape[0], w.shape[1]), f32)], axis=0)
    temp, iaw, anom, hf = _tc_heads(
        s, c, Wqkv[2 * EMB:], bqkv[2 * EMB:].reshape(1, -1), Wo,
        bo.reshape(1, -1), Tw1, Tb1.reshape(1, -1), Tw2, Tb2.reshape(1, -1),
        pad8(Tw3), Tb3.reshape(1, -1), Iw1, Ib1.reshape(1, -1), Iw2,
        Ib2.reshape(1, -1), pad8(Iw3), Ib3.reshape(1, -1), Aw1,
        Ab1.reshape(1, -1),
        pad8(Aw2), jnp.concatenate([Ab2, jnp.zeros((6,), f32)]).reshape(1, -1))
    return (temp, iaw, anom, hf)


# speedup vs baseline: 1.2270x; 1.2270x over previous
import functools

import jax
import jax.numpy as jnp
from jax import lax
from jax.experimental import pallas as pl
from jax.experimental.pallas import tpu as pltpu
from jax.experimental.pallas import tpu_sc as plsc

N = 50000
E = 800000
IN = 128
HID = 64
EMB = 32
G = 64

NC = 2
NS = 16
NW = NC * NS
CH = 256
NCHUNK = 98
SB_CH = 14
SB_E = SB_CH * CH
NSB = NCHUNK // SB_CH
EPT = NCHUNK * CH
E_PAD = NW * EPT
NA = 50176
TPT = NA // NS
BLK = 3584
NBLK = NA // BLK
NAP = NA // 4
BLKP = BLK // 4

_MESH = plsc.VectorSubcoreMesh(
    core_axis_name="c", subcore_axis_name="s", num_cores=NC, num_subcores=NS)


def _sc_prop_body(table, ep, zeros, out, src_v, dst_v, rows0, rows1,
                  acc, gsem0, gsem1, ssem0, ssem1):
    c = lax.axis_index("c")
    s = lax.axis_index("s")
    wid = s * NC + c
    pltpu.sync_copy(zeros.at[pl.ds(s * TPT, TPT)], acc.at[pl.ds(s * TPT, TPT)])
    plsc.subcore_barrier()

    @pl.loop(0, NSB)
    def _(b):
        base = wid * EPT + b * SB_E
        pltpu.sync_copy(ep.at[0, pl.ds(base, SB_E)], src_v)
        pltpu.sync_copy(ep.at[1, pl.ds(base, SB_E)], dst_v)
        pltpu.async_copy(table.at[src_v.at[pl.ds(0, CH)]], rows0, gsem0)

        @pl.loop(0, SB_CH, step=2)
        def _(j):
            pltpu.async_copy(
                table.at[src_v.at[pl.ds((j + 1) * CH, CH)]], rows1, gsem1)
            pltpu.make_async_copy(
                table.at[src_v.at[pl.ds(j * CH, CH)]], rows0, gsem0).wait()
            pltpu.sync_copy(rows0, acc.at[dst_v.at[pl.ds(j * CH, CH)]],
                            add=True)

            @pl.when(j + 2 < SB_CH)
            def _():
                pltpu.async_copy(
                    table.at[src_v.at[pl.ds((j + 2) * CH, CH)]], rows0, gsem0)

            pltpu.make_async_copy(
                table.at[src_v.at[pl.ds((j + 1) * CH, CH)]], rows1,
                gsem1).wait()
            pltpu.sync_copy(rows1, acc.at[dst_v.at[pl.ds((j + 1) * CH, CH)]],
                            add=True)

    plsc.subcore_barrier()
    pltpu.sync_copy(acc.at[pl.ds(s * TPT, TPT)], out.at[c, pl.ds(s * TPT, TPT)])


_sc_prop = pl.kernel(
    _sc_prop_body,
    out_type=jax.ShapeDtypeStruct((NC, NA, EMB), jnp.float32),
    mesh=_MESH,
    compiler_params=pltpu.CompilerParams(use_tc_tiling_on_sc=False),
    scratch_types=[
        pltpu.VMEM((SB_E,), jnp.int32),
        pltpu.VMEM((SB_E,), jnp.int32),
        pltpu.VMEM((CH, EMB), jnp.float32),
        pltpu.VMEM((CH, EMB), jnp.float32),
        pltpu.VMEM_SHARED((NA, EMB), jnp.float32),
        pltpu.SemaphoreType.DMA,
        pltpu.SemaphoreType.DMA,
        pltpu.SemaphoreType.DMA,
        pltpu.SemaphoreType.DMA,
    ],
)


def _sc_deg_body(ep, ones, zeros8, out, dst_v, ones_v, acc, sem):
    c = lax.axis_index("c")
    s = lax.axis_index("s")
    wid = s * NC + c
    pltpu.sync_copy(zeros8.at[pl.ds(s * TPT, TPT)], acc.at[pl.ds(s * TPT, TPT)])
    pltpu.sync_copy(ones, ones_v)
    plsc.subcore_barrier()

    @pl.loop(0, NSB)
    def _(b):
        pltpu.sync_copy(ep.at[1, pl.ds(wid * EPT + b * SB_E, SB_E)], dst_v)

        @pl.loop(0, SB_CH)
        def _(j):
            pltpu.async_copy(ones_v, acc.at[dst_v.at[pl.ds(j * CH, CH)]],
                             sem, add=True)

        @pl.loop(0, SB_CH)
        def _(j):
            pltpu.make_async_copy(
                ones_v, acc.at[dst_v.at[pl.ds(j * CH, CH)]], sem).wait()

    plsc.subcore_barrier()
    pltpu.sync_copy(acc.at[pl.ds(s * TPT, TPT)], out.at[c, pl.ds(s * TPT, TPT)])


_sc_deg = pl.kernel(
    _sc_deg_body,
    out_type=jax.ShapeDtypeStruct((NC, NA, 8), jnp.float32),
    mesh=_MESH,
    compiler_params=pltpu.CompilerParams(use_tc_tiling_on_sc=False),
    scratch_types=[
        pltpu.VMEM((SB_E,), jnp.int32),
        pltpu.VMEM((CH, 8), jnp.float32),
        pltpu.VMEM_SHARED((NA, 8), jnp.float32),
        pltpu.SemaphoreType.DMA,
    ],
)


def _mm(a, b):
    return lax.dot_general(a, b, (((1,), (0,)), ((), ())),
                           preferred_element_type=jnp.float32)


def _stage0_body(xp_ref, wep_ref, bep_ref, degp_ref, seld_ref, t1_ref,
                 dinv_ref):
    i = pl.program_id(0)
    h0 = _mm(xp_ref[...], wep_ref[...]) + bep_ref[...]
    deg8 = degp_ref[0] + degp_ref[1]
    deg = _mm(deg8, seld_ref[...]) + 1.0
    rows = lax.broadcasted_iota(jnp.int32, (BLKP, 128), 0)
    lanes = lax.broadcasted_iota(jnp.int32, (BLKP, 128), 1)
    node = 4 * (i * BLKP + rows) + lanes // EMB
    dinv = jnp.where(node < N, lax.rsqrt(deg), 0.0)
    t1_ref[...] = dinv * h0
    dinv_ref[...] = dinv


def _tc_stage0(xp, WeP, beP, degp4, SelD):
    return pl.pallas_call(
        _stage0_body,
        grid=(NBLK,),
        in_specs=[
            pl.BlockSpec((BLKP, 4 * IN), lambda i: (i, 0)),
            pl.BlockSpec((4 * IN, 128), lambda i: (0, 0)),
            pl.BlockSpec((1, 128), lambda i: (0, 0)),
            pl.BlockSpec((NC, BLKP, EMB), lambda i: (0, i, 0)),
            pl.BlockSpec((EMB, 128), lambda i: (0, 0)),
        ],
        out_specs=[
            pl.BlockSpec((BLKP, 128), lambda i: (i, 0)),
            pl.BlockSpec((BLKP, 128), lambda i: (i, 0)),
        ],
        out_shape=[
            jax.ShapeDtypeStruct((NAP, 128), jnp.float32),
            jax.ShapeDtypeStruct((NAP, 128), jnp.float32),
        ],
    )(xp, WeP, beP, degp4, SelD)


def _layer1_body(p_ref, t1_ref, dinv_ref, w1_ref, b1_ref, sela_ref, selb_ref,
                 a_ref, b_ref):
    dinv = dinv_ref[...]
    pre = dinv * (p_ref[0] + p_ref[1] + t1_ref[...])
    h1 = jax.nn.relu(_mm(pre, w1_ref[...]) + b1_ref[...])
    a_ref[...] = dinv * _mm(h1, sela_ref[...])
    b_ref[...] = dinv * _mm(h1, selb_ref[...])


def _tc_layer1(P14, t1p, dinvp, W1P, b1P, SelA, SelB):
    return pl.pallas_call(
        _layer1_body,
        grid=(NBLK,),
        in_specs=[
            pl.BlockSpec((NC, BLKP, 128), lambda i: (0, i, 0)),
            pl.BlockSpec((BLKP, 128), lambda i: (i, 0)),
            pl.BlockSpec((BLKP, 128), lambda i: (i, 0)),
            pl.BlockSpec((128, 256), lambda i: (0, 0)),
            pl.BlockSpec((1, 256), lambda i: (0, 0)),
            pl.BlockSpec((256, 128), lambda i: (0, 0)),
            pl.BlockSpec((256, 128), lambda i: (0, 0)),
        ],
        out_specs=[
            pl.BlockSpec((BLKP, 128), lambda i: (i, 0)),
            pl.BlockSpec((BLKP, 128), lambda i: (i, 0)),
        ],
        out_shape=[
            jax.ShapeDtypeStruct((NAP, 128), jnp.float32),
            jax.ShapeDtypeStruct((NAP, 128), jnp.float32),
        ],
    )(P14, t1p, dinvp, W1P, b1P, SelA, SelB)


def _layer2_body(pa_ref, pb_ref, ta_ref, tb_ref, dinv_ref, w2a_ref, w2b_ref,
                 b2_ref, w3_ref, t3_ref):
    dinv = dinv_ref[...]
    col_a = dinv * (pa_ref[0] + pa_ref[1] + ta_ref[...])
    col_b = dinv * (pb_ref[0] + pb_ref[1] + tb_ref[...])
    h2 = jax.nn.relu(_mm(col_a, w2a_ref[...]) + _mm(col_b, w2b_ref[...])
                     + b2_ref[...])
    t3_ref[...] = dinv * _mm(h2, w3_ref[...])


def _tc_layer2(P2a4, P2b4, t2ap, t2bp, dinvp, W2aP, W2bP, b2P, W3P):
    return pl.pallas_call(
        _layer2_body,
        grid=(NBLK,),
        in_specs=[
            pl.BlockSpec((NC, BLKP, 128), lambda i: (0, i, 0)),
            pl.BlockSpec((NC, BLKP, 128), lambda i: (0, i, 0)),
            pl.BlockSpec((BLKP, 128), lambda i: (i, 0)),
            pl.BlockSpec((BLKP, 128), lambda i: (i, 0)),
            pl.BlockSpec((BLKP, 128), lambda i: (i, 0)),
            pl.BlockSpec((128, 256), lambda i: (0, 0)),
            pl.BlockSpec((128, 256), lambda i: (0, 0)),
            pl.BlockSpec((1, 256), lambda i: (0, 0)),
            pl.BlockSpec((256, 128), lambda i: (0, 0)),
        ],
        out_specs=[pl.BlockSpec((BLKP, 128), lambda i: (i, 0))],
        out_shape=[jax.ShapeDtypeStruct((NAP, 128), jnp.float32)],
    )(P2a4, P2b4, t2ap, t2bp, dinvp, W2aP, W2bP, b2P, W3P)[0]


def _pool_body(p_ref, t3_ref, dinv_ref, b3_ref, batch_ref, s_ref, c_ref):
    i = pl.program_id(0)
    out3 = dinv_ref[...] * (p_ref[0] + p_ref[1] + t3_ref[...]) + b3_ref[...]
    iota_g = lax.broadcasted_iota(jnp.int32, (G, BLKP), 0)
    ones_b = jnp.ones((BLKP, EMB), jnp.float32)
    s_part = jnp.zeros((G, EMB), jnp.float32)
    c_part = jnp.zeros((G, EMB), jnp.float32)
    for q in range(4):
        oh = jnp.where(batch_ref[q:q + 1, :] == iota_g, 1.0, 0.0)
        s_part += _mm(oh, out3[:, EMB * q:EMB * (q + 1)])
        c_part += _mm(oh, ones_b)

    @pl.when(i == 0)
    def _():
        s_ref[...] = jnp.zeros_like(s_ref)
        c_ref[...] = jnp.zeros_like(c_ref)

    s_ref[...] += s_part
    c_ref[...] += c_part


def _tc_pool(P34, t3p, dinvp, b3P, batchp4):
    return pl.pallas_call(
        _pool_body,
        grid=(NBLK,),
        in_specs=[
            pl.BlockSpec((NC, BLKP, 128), lambda i: (0, i, 0)),
            pl.BlockSpec((BLKP, 128), lambda i: (i, 0)),
            pl.BlockSpec((BLKP, 128), lambda i: (i, 0)),
            pl.BlockSpec((1, 128), lambda i: (0, 0)),
            pl.BlockSpec((8, BLKP), lambda i: (0, i)),
        ],
        out_specs=[
            pl.BlockSpec((G, EMB), lambda i: (0, 0)),
            pl.BlockSpec((G, EMB), lambda i: (0, 0)),
        ],
        out_shape=[
            jax.ShapeDtypeStruct((G, EMB), jnp.float32),
            jax.ShapeDtypeStruct((G, EMB), jnp.float32),
        ],
    )(P34, t3p, dinvp, b3P, batchp4)


def _dotT(a, w):
    return lax.dot_general(a, w, (((1,), (1,)), ((), ())),
                           preferred_element_type=jnp.float32)


def _heads_body(s_ref, c_ref, wv_ref, bv_ref, wo_ref, bo_ref,
                tw1_ref, tb1_ref, tw2_ref, tb2_ref, tw3_ref, tb3_ref,
                iw1_ref, ib1_ref, iw2_ref, ib2_ref, iw3_ref, ib3_ref,
                aw1_ref, ab1_ref, aw2_ref, ab2_ref,
                temp_ref, iaw_ref, anom_ref, hf_ref):
    mean = s_ref[...] / jnp.maximum(c_ref[...], 1.0)
    v = _dotT(mean, wv_ref[...]) + bv_ref[...]
    hf = _dotT(v, wo_ref[...]) + bo_ref[...]
    hf_ref[...] = hf
    t = jax.nn.relu(_dotT(hf, tw1_ref[...]) + tb1_ref[...])
    t = jax.nn.relu(_dotT(t, tw2_ref[...]) + tb2_ref[...])
    temp_ref[...] = jax.nn.sigmoid(_dotT(t, tw3_ref[...]) + tb3_ref[0, 0])[:, 0:1]
    w = jax.nn.relu(_dotT(hf, iw1_ref[...]) + ib1_ref[...])
    w = jax.nn.relu(_dotT(w, iw2_ref[...]) + ib2_ref[...])
    iaw_ref[...] = jax.nn.relu(_dotT(w, iw3_ref[...]) + ib3_ref[0, 0])[:, 0:1]
    a = jax.nn.relu(_dotT(hf, aw1_ref[...]) + ab1_ref[...])
    lg = _dotT(a, aw2_ref[...]) + ab2_ref[...]
    l0, l1 = lg[:, 0:1], lg[:, 1:2]
    m = jnp.maximum(l0, l1)
    e0, e1 = jnp.exp(l0 - m), jnp.exp(l1 - m)
    tot = e0 + e1
    anom_ref[...] = jnp.concatenate([e0 / tot, e1 / tot], axis=1)


def _tc_heads(s, c, Wv, bv2, Wo, bo2, Tw1, Tb12, Tw2, Tb22, Tw3, Tb32,
              Iw1, Ib12, Iw2, Ib22, Iw3, Ib32, Aw1, Ab12, Aw2, Ab22):
    return pl.pallas_call(
        _heads_body,
        out_shape=[
            jax.ShapeDtypeStruct((G, 1), jnp.float32),
            jax.ShapeDtypeStruct((G, 1), jnp.float32),
            jax.ShapeDtypeStruct((G, 2), jnp.float32),
            jax.ShapeDtypeStruct((G, EMB), jnp.float32),
        ],
    )(s, c, Wv, bv2, Wo, bo2, Tw1, Tb12, Tw2, Tb22, Tw3, Tb32,
      Iw1, Ib12, Iw2, Ib22, Iw3, Ib32, Aw1, Ab12, Aw2, Ab22)


def kernel(x, edge_index, batch, We, be, W1, b1, W2, b2, W3, b3, Wqkv, bqkv,
           Wo, bo, Tw1, Tb1, Tw2, Tb2, Tw3, Tb3, Iw1, Ib1, Iw2, Ib2, Iw3, Ib3,
           Aw1, Ab1, Aw2, Ab2):
    f32 = jnp.float32
    eye4 = jnp.eye(4, dtype=f32)
    kron = jnp.kron
    ep = jnp.pad(edge_index, ((0, 0), (0, E_PAD - E)), constant_values=N)
    batch_pad = jnp.concatenate([batch, jnp.full((NA - N,), G, jnp.int32)])
    batchp4 = jnp.concatenate(
        [batch_pad.reshape(NAP, 4).T, jnp.full((4, NAP), G, jnp.int32)], axis=0)
    zeros32 = jnp.zeros((NA, EMB), f32)
    zeros8 = jnp.zeros((NA, 8), f32)
    ones8 = jnp.ones((CH, 8), f32)

    xp = jnp.concatenate(
        [x.reshape(N // 4, 4 * IN),
         jnp.zeros((NAP - N // 4, 4 * IN), f32)])
    WeP = kron(eye4, We.T)
    beP = jnp.tile(be, 4).reshape(1, 128)
    SelD = kron(eye4, jnp.full((8, EMB), 0.125, f32))
    W1P = kron(eye4, W1.T)
    b1P = jnp.tile(b1, 4).reshape(1, 256)
    SelA = kron(eye4, jnp.eye(HID, EMB, dtype=f32))
    SelB = kron(eye4, jnp.eye(HID, EMB, k=-EMB, dtype=f32))
    W2aP = kron(eye4, W2[:, :EMB].T)
    W2bP = kron(eye4, W2[:, EMB:].T)
    b2P = jnp.tile(b2, 4).reshape(1, 256)
    W3P = kron(eye4, W3.T)
    b3P = jnp.tile(b3, 4).reshape(1, 128)

    degp = _sc_deg(ep, ones8, zeros8)
    t1p, dinvp = _tc_stage0(xp, WeP, beP, degp.reshape(NC, NAP, EMB), SelD)
    P1 = _sc_prop(t1p.reshape(NA, EMB), ep, zeros32)
    t2ap, t2bp = _tc_layer1(P1.reshape(NC, NAP, 128), t1p, dinvp, W1P, b1P,
                            SelA, SelB)
    P2a = _sc_prop(t2ap.reshape(NA, EMB), ep, zeros32)
    P2b = _sc_prop(t2bp.reshape(NA, EMB), ep, zeros32)
    t3p = _tc_layer2(P2a.reshape(NC, NAP, 128), P2b.reshape(NC, NAP, 128),
                     t2ap, t2bp, dinvp, W2aP, W2bP, b2P, W3P)
    P3 = _sc_prop(t3p.reshape(NA, EMB), ep, zeros32)
    s, c = _tc_pool(P3.reshape(NC, NAP, 128), t3p, dinvp, b3P, batchp4)
    pad8 = lambda w: jnp.concatenate(
        [w, jnp.zeros((8 - w.shape[0], w.shape[1]), f32)], axis=0)
    temp, iaw, anom, hf = _tc_heads(
        s, c, Wqkv[2 * EMB:], bqkv[2 * EMB:].reshape(1, -1), Wo,
        bo.reshape(1, -1), Tw1, Tb1.reshape(1, -1), Tw2, Tb2.reshape(1, -1),
        pad8(Tw3), Tb3.reshape(1, -1), Iw1, Ib1.reshape(1, -1), Iw2,
        Ib2.reshape(1, -1), pad8(Iw3), Ib3.reshape(1, -1), Aw1,
        Ab1.reshape(1, -1),
        pad8(Aw2), jnp.concatenate([Ab2, jnp.zeros((6,), f32)]).reshape(1, -1))
    return (temp, iaw, anom, hf)


# speedup vs baseline: 1.2668x; 1.0325x over previous
import functools

import jax
import jax.numpy as jnp
from jax import lax
from jax.experimental import pallas as pl
from jax.experimental.pallas import tpu as pltpu
from jax.experimental.pallas import tpu_sc as plsc

N = 50000
E = 800000
IN = 128
HID = 64
EMB = 32
G = 64

NC = 2
NS = 16
NW = NC * NS
CH = 256
SB_CH = 14
SB_E = SB_CH * CH
PAIR_E = 50176
C0_E = 8 * SB_E
E_PAD = NS * PAIR_E
TAIL_BASE = E_PAD - SB_E
NA = 50176
TPT = NA // NS
BLK = 3584
NBLK = NA // BLK
NAP = NA // 4
BLKP = BLK // 4

_MESH = plsc.VectorSubcoreMesh(
    core_axis_name="c", subcore_axis_name="s", num_cores=NC, num_subcores=NS)


def _sc_prop_body(table, ep, tail, zeros, out, src_v, dst_v, rows0, rows1,
                  acc, gsem0, gsem1):
    c = lax.axis_index("c")
    s = lax.axis_index("s")
    pltpu.sync_copy(zeros.at[pl.ds(s * TPT, TPT)], acc.at[pl.ds(s * TPT, TPT)])
    plsc.subcore_barrier()

    @pl.loop(0, 8 - 2 * c)
    def _(b):
        base = s * PAIR_E + c * C0_E + b * SB_E

        @pl.when(base < TAIL_BASE)
        def _():
            pltpu.sync_copy(ep.at[0, pl.ds(base, SB_E)], src_v)
            pltpu.sync_copy(ep.at[1, pl.ds(base, SB_E)], dst_v)

        @pl.when(base >= TAIL_BASE)
        def _():
            pltpu.sync_copy(tail.at[0], src_v)
            pltpu.sync_copy(tail.at[1], dst_v)
        pltpu.async_copy(table.at[src_v.at[pl.ds(0, CH)]], rows0, gsem0)

        @pl.loop(0, SB_CH, step=2)
        def _(j):
            pltpu.async_copy(
                table.at[src_v.at[pl.ds((j + 1) * CH, CH)]], rows1, gsem1)
            pltpu.make_async_copy(
                table.at[src_v.at[pl.ds(j * CH, CH)]], rows0, gsem0).wait()
            pltpu.sync_copy(rows0, acc.at[dst_v.at[pl.ds(j * CH, CH)]],
                            add=True)

            @pl.when(j + 2 < SB_CH)
            def _():
                pltpu.async_copy(
                    table.at[src_v.at[pl.ds((j + 2) * CH, CH)]], rows0, gsem0)

            pltpu.make_async_copy(
                table.at[src_v.at[pl.ds((j + 1) * CH, CH)]], rows1,
                gsem1).wait()
            pltpu.sync_copy(rows1, acc.at[dst_v.at[pl.ds((j + 1) * CH, CH)]],
                            add=True)

    plsc.subcore_barrier()
    pltpu.sync_copy(acc.at[pl.ds(s * TPT, TPT)], out.at[c, pl.ds(s * TPT, TPT)])


_sc_prop = pl.kernel(
    _sc_prop_body,
    out_type=jax.ShapeDtypeStruct((NC, NA, EMB), jnp.float32),
    mesh=_MESH,
    compiler_params=pltpu.CompilerParams(use_tc_tiling_on_sc=False),
    scratch_types=[
        pltpu.VMEM((SB_E,), jnp.int32),
        pltpu.VMEM((SB_E,), jnp.int32),
        pltpu.VMEM((CH, EMB), jnp.float32),
        pltpu.VMEM((CH, EMB), jnp.float32),
        pltpu.VMEM_SHARED((NA, EMB), jnp.float32),
        pltpu.SemaphoreType.DMA,
        pltpu.SemaphoreType.DMA,
    ],
)


def _sc_deg_body(ep, tail, ones, zeros8, out, dst_v, ones_v, acc, sem):
    c = lax.axis_index("c")
    s = lax.axis_index("s")
    pltpu.sync_copy(zeros8.at[pl.ds(s * TPT, TPT)], acc.at[pl.ds(s * TPT, TPT)])
    pltpu.sync_copy(ones, ones_v)
    plsc.subcore_barrier()

    @pl.loop(0, 8 - 2 * c)
    def _(b):
        base = s * PAIR_E + c * C0_E + b * SB_E

        @pl.when(base < TAIL_BASE)
        def _():
            pltpu.sync_copy(ep.at[1, pl.ds(base, SB_E)], dst_v)

        @pl.when(base >= TAIL_BASE)
        def _():
            pltpu.sync_copy(tail.at[1], dst_v)

        @pl.loop(0, SB_CH)
        def _(j):
            pltpu.async_copy(ones_v, acc.at[dst_v.at[pl.ds(j * CH, CH)]],
                             sem, add=True)

        @pl.loop(0, SB_CH)
        def _(j):
            pltpu.make_async_copy(
                ones_v, acc.at[dst_v.at[pl.ds(j * CH, CH)]], sem).wait()

    plsc.subcore_barrier()
    pltpu.sync_copy(acc.at[pl.ds(s * TPT, TPT)], out.at[c, pl.ds(s * TPT, TPT)])


_sc_deg = pl.kernel(
    _sc_deg_body,
    out_type=jax.ShapeDtypeStruct((NC, NA, 8), jnp.float32),
    mesh=_MESH,
    compiler_params=pltpu.CompilerParams(use_tc_tiling_on_sc=False),
    scratch_types=[
        pltpu.VMEM((SB_E,), jnp.int32),
        pltpu.VMEM((CH, 8), jnp.float32),
        pltpu.VMEM_SHARED((NA, 8), jnp.float32),
        pltpu.SemaphoreType.DMA,
    ],
)


def _mm(a, b):
    return lax.dot_general(a, b, (((1,), (0,)), ((), ())),
                           preferred_element_type=jnp.float32)


def _stage0_body(xp_ref, wep_ref, bep_ref, degp_ref, seld_ref, t1_ref,
                 dinv_ref):
    i = pl.program_id(0)
    h0 = _mm(xp_ref[...], wep_ref[...]) + bep_ref[...]
    deg8 = degp_ref[0] + degp_ref[1]
    deg = _mm(deg8, seld_ref[...]) + 1.0
    rows = lax.broadcasted_iota(jnp.int32, (BLKP, 128), 0)
    lanes = lax.broadcasted_iota(jnp.int32, (BLKP, 128), 1)
    node = 4 * (i * BLKP + rows) + lanes // EMB
    dinv = jnp.where(node < N, lax.rsqrt(deg), 0.0)
    t1_ref[...] = dinv * h0
    dinv_ref[...] = dinv


def _tc_stage0(xp, WeP, beP, degp4, SelD):
    return pl.pallas_call(
        _stage0_body,
        grid=(NBLK,),
        in_specs=[
            pl.BlockSpec((BLKP, 4 * IN), lambda i: (i, 0)),
            pl.BlockSpec((4 * IN, 128), lambda i: (0, 0)),
            pl.BlockSpec((1, 128), lambda i: (0, 0)),
            pl.BlockSpec((NC, BLKP, EMB), lambda i: (0, i, 0)),
            pl.BlockSpec((EMB, 128), lambda i: (0, 0)),
        ],
        out_specs=[
            pl.BlockSpec((BLKP, 128), lambda i: (i, 0)),
            pl.BlockSpec((BLKP, 128), lambda i: (i, 0)),
        ],
        out_shape=[
            jax.ShapeDtypeStruct((NAP, 128), jnp.float32),
            jax.ShapeDtypeStruct((NAP, 128), jnp.float32),
        ],
    )(xp, WeP, beP, degp4, SelD)


def _layer1_body(p_ref, t1_ref, dinv_ref, w1_ref, b1_ref, sela_ref, selb_ref,
                 a_ref, b_ref):
    dinv = dinv_ref[...]
    pre = dinv * (p_ref[0] + p_ref[1] + t1_ref[...])
    h1 = jax.nn.relu(_mm(pre, w1_ref[...]) + b1_ref[...])
    a_ref[...] = dinv * _mm(h1, sela_ref[...])
    b_ref[...] = dinv * _mm(h1, selb_ref[...])


def _tc_layer1(P14, t1p, dinvp, W1P, b1P, SelA, SelB):
    return pl.pallas_call(
        _layer1_body,
        grid=(NBLK,),
        in_specs=[
            pl.BlockSpec((NC, BLKP, 128), lambda i: (0, i, 0)),
            pl.BlockSpec((BLKP, 128), lambda i: (i, 0)),
            pl.BlockSpec((BLKP, 128), lambda i: (i, 0)),
            pl.BlockSpec((128, 256), lambda i: (0, 0)),
            pl.BlockSpec((1, 256), lambda i: (0, 0)),
            pl.BlockSpec((256, 128), lambda i: (0, 0)),
            pl.BlockSpec((256, 128), lambda i: (0, 0)),
        ],
        out_specs=[
            pl.BlockSpec((BLKP, 128), lambda i: (i, 0)),
            pl.BlockSpec((BLKP, 128), lambda i: (i, 0)),
        ],
        out_shape=[
            jax.ShapeDtypeStruct((NAP, 128), jnp.float32),
            jax.ShapeDtypeStruct((NAP, 128), jnp.float32),
        ],
    )(P14, t1p, dinvp, W1P, b1P, SelA, SelB)


def _layer2_body(pa_ref, pb_ref, ta_ref, tb_ref, dinv_ref, w2a_ref, w2b_ref,
                 b2_ref, w3_ref, t3_ref):
    dinv = dinv_ref[...]
    col_a = dinv * (pa_ref[0] + pa_ref[1] + ta_ref[...])
    col_b = dinv * (pb_ref[0] + pb_ref[1] + tb_ref[...])
    h2 = jax.nn.relu(_mm(col_a, w2a_ref[...]) + _mm(col_b, w2b_ref[...])
                     + b2_ref[...])
    t3_ref[...] = dinv * _mm(h2, w3_ref[...])


def _tc_layer2(P2a4, P2b4, t2ap, t2bp, dinvp, W2aP, W2bP, b2P, W3P):
    return pl.pallas_call(
        _layer2_body,
        grid=(NBLK,),
        in_specs=[
            pl.BlockSpec((NC, BLKP, 128), lambda i: (0, i, 0)),
            pl.BlockSpec((NC, BLKP, 128), lambda i: (0, i, 0)),
            pl.BlockSpec((BLKP, 128), lambda i: (i, 0)),
            pl.BlockSpec((BLKP, 128), lambda i: (i, 0)),
            pl.BlockSpec((BLKP, 128), lambda i: (i, 0)),
            pl.BlockSpec((128, 256), lambda i: (0, 0)),
            pl.BlockSpec((128, 256), lambda i: (0, 0)),
            pl.BlockSpec((1, 256), lambda i: (0, 0)),
            pl.BlockSpec((256, 128), lambda i: (0, 0)),
        ],
        out_specs=[pl.BlockSpec((BLKP, 128), lambda i: (i, 0))],
        out_shape=[jax.ShapeDtypeStruct((NAP, 128), jnp.float32)],
    )(P2a4, P2b4, t2ap, t2bp, dinvp, W2aP, W2bP, b2P, W3P)[0]


def _pool_body(p_ref, t3_ref, dinv_ref, b3_ref, batch_ref, s_ref, c_ref):
    i = pl.program_id(0)
    out3 = dinv_ref[...] * (p_ref[0] + p_ref[1] + t3_ref[...]) + b3_ref[...]
    iota_g = lax.broadcasted_iota(jnp.int32, (G, BLKP), 0)
    ones_b = jnp.ones((BLKP, EMB), jnp.float32)
    s_part = jnp.zeros((G, EMB), jnp.float32)
    c_part = jnp.zeros((G, EMB), jnp.float32)
    for q in range(4):
        oh = jnp.where(batch_ref[q:q + 1, :] == iota_g, 1.0, 0.0)
        s_part += _mm(oh, out3[:, EMB * q:EMB * (q + 1)])
        c_part += _mm(oh, ones_b)

    @pl.when(i == 0)
    def _():
        s_ref[...] = jnp.zeros_like(s_ref)
        c_ref[...] = jnp.zeros_like(c_ref)

    s_ref[...] += s_part
    c_ref[...] += c_part


def _tc_pool(P34, t3p, dinvp, b3P, batchp4):
    return pl.pallas_call(
        _pool_body,
        grid=(NBLK,),
        in_specs=[
            pl.BlockSpec((NC, BLKP, 128), lambda i: (0, i, 0)),
            pl.BlockSpec((BLKP, 128), lambda i: (i, 0)),
            pl.BlockSpec((BLKP, 128), lambda i: (i, 0)),
            pl.BlockSpec((1, 128), lambda i: (0, 0)),
            pl.BlockSpec((8, BLKP), lambda i: (0, i)),
        ],
        out_specs=[
            pl.BlockSpec((G, EMB), lambda i: (0, 0)),
            pl.BlockSpec((G, EMB), lambda i: (0, 0)),
        ],
        out_shape=[
            jax.ShapeDtypeStruct((G, EMB), jnp.float32),
            jax.ShapeDtypeStruct((G, EMB), jnp.float32),
        ],
    )(P34, t3p, dinvp, b3P, batchp4)


def _dotT(a, w):
    return lax.dot_general(a, w, (((1,), (1,)), ((), ())),
                           preferred_element_type=jnp.float32)


def _heads_body(s_ref, c_ref, wv_ref, bv_ref, wo_ref, bo_ref,
                tw1_ref, tb1_ref, tw2_ref, tb2_ref, tw3_ref, tb3_ref,
                iw1_ref, ib1_ref, iw2_ref, ib2_ref, iw3_ref, ib3_ref,
                aw1_ref, ab1_ref, aw2_ref, ab2_ref,
                temp_ref, iaw_ref, anom_ref, hf_ref):
    mean = s_ref[...] / jnp.maximum(c_ref[...], 1.0)
    v = _dotT(mean, wv_ref[...]) + bv_ref[...]
    hf = _dotT(v, wo_ref[...]) + bo_ref[...]
    hf_ref[...] = hf
    t = jax.nn.relu(_dotT(hf, tw1_ref[...]) + tb1_ref[...])
    t = jax.nn.relu(_dotT(t, tw2_ref[...]) + tb2_ref[...])
    temp_ref[...] = jax.nn.sigmoid(_dotT(t, tw3_ref[...]) + tb3_ref[0, 0])[:, 0:1]
    w = jax.nn.relu(_dotT(hf, iw1_ref[...]) + ib1_ref[...])
    w = jax.nn.relu(_dotT(w, iw2_ref[...]) + ib2_ref[...])
    iaw_ref[...] = jax.nn.relu(_dotT(w, iw3_ref[...]) + ib3_ref[0, 0])[:, 0:1]
    a = jax.nn.relu(_dotT(hf, aw1_ref[...]) + ab1_ref[...])
    lg = _dotT(a, aw2_ref[...]) + ab2_ref[...]
    l0, l1 = lg[:, 0:1], lg[:, 1:2]
    m = jnp.maximum(l0, l1)
    e0, e1 = jnp.exp(l0 - m), jnp.exp(l1 - m)
    tot = e0 + e1
    anom_ref[...] = jnp.concatenate([e0 / tot, e1 / tot], axis=1)


def _tc_heads(s, c, Wv, bv2, Wo, bo2, Tw1, Tb12, Tw2, Tb22, Tw3, Tb32,
              Iw1, Ib12, Iw2, Ib22, Iw3, Ib32, Aw1, Ab12, Aw2, Ab22):
    return pl.pallas_call(
        _heads_body,
        out_shape=[
            jax.ShapeDtypeStruct((G, 1), jnp.float32),
            jax.ShapeDtypeStruct((G, 1), jnp.float32),
            jax.ShapeDtypeStruct((G, 2), jnp.float32),
            jax.ShapeDtypeStruct((G, EMB), jnp.float32),
        ],
    )(s, c, Wv, bv2, Wo, bo2, Tw1, Tb12, Tw2, Tb22, Tw3, Tb32,
      Iw1, Ib12, Iw2, Ib22, Iw3, Ib32, Aw1, Ab12, Aw2, Ab22)


def kernel(x, edge_index, batch, We, be, W1, b1, W2, b2, W3, b3, Wqkv, bqkv,
           Wo, bo, Tw1, Tb1, Tw2, Tb2, Tw3, Tb3, Iw1, Ib1, Iw2, Ib2, Iw3, Ib3,
           Aw1, Ab1, Aw2, Ab2):
    f32 = jnp.float32
    eye4 = jnp.eye(4, dtype=f32)
    kron = jnp.kron
    tailE = jnp.concatenate(
        [edge_index[:, TAIL_BASE:], jnp.full((2, E_PAD - E), N, jnp.int32)],
        axis=1)
    batch_pad = jnp.concatenate([batch, jnp.full((NA - N,), G, jnp.int32)])
    batchp4 = jnp.concatenate(
        [batch_pad.reshape(NAP, 4).T, jnp.full((4, NAP), G, jnp.int32)], axis=0)
    zeros32 = jnp.zeros((NA, EMB), f32)
    zeros8 = jnp.zeros((NA, 8), f32)
    ones8 = jnp.ones((CH, 8), f32)

    xp = jnp.concatenate(
        [x.reshape(N // 4, 4 * IN),
         jnp.zeros((NAP - N // 4, 4 * IN), f32)])
    WeP = kron(eye4, We.T)
    beP = jnp.tile(be, 4).reshape(1, 128)
    SelD = kron(eye4, jnp.full((8, EMB), 0.125, f32))
    W1P = kron(eye4, W1.T)
    b1P = jnp.tile(b1, 4).reshape(1, 256)
    SelA = kron(eye4, jnp.eye(HID, EMB, dtype=f32))
    SelB = kron(eye4, jnp.eye(HID, EMB, k=-EMB, dtype=f32))
    W2aP = kron(eye4, W2[:, :EMB].T)
    W2bP = kron(eye4, W2[:, EMB:].T)
    b2P = jnp.tile(b2, 4).reshape(1, 256)
    W3P = kron(eye4, W3.T)
    b3P = jnp.tile(b3, 4).reshape(1, 128)

    degp = _sc_deg(edge_index, tailE, ones8, zeros8)
    t1p, dinvp = _tc_stage0(xp, WeP, beP, degp.reshape(NC, NAP, EMB), SelD)
    P1 = _sc_prop(t1p.reshape(NA, EMB), edge_index, tailE, zeros32)
    t2ap, t2bp = _tc_layer1(P1.reshape(NC, NAP, 128), t1p, dinvp, W1P, b1P,
                            SelA, SelB)
    P2a = _sc_prop(t2ap.reshape(NA, EMB), edge_index, tailE, zeros32)
    P2b = _sc_prop(t2bp.reshape(NA, EMB), edge_index, tailE, zeros32)
    t3p = _tc_layer2(P2a.reshape(NC, NAP, 128), P2b.reshape(NC, NAP, 128),
                     t2ap, t2bp, dinvp, W2aP, W2bP, b2P, W3P)
    P3 = _sc_prop(t3p.reshape(NA, EMB), edge_index, tailE, zeros32)
    s, c = _tc_pool(P3.reshape(NC, NAP, 128), t3p, dinvp, b3P, batchp4)
    pad8 = lambda w: jnp.concatenate(
        [w, jnp.zeros((8 - w.shape[0], w.shape[1]), f32)], axis=0)
    temp, iaw, anom, hf = _tc_heads(
        s, c, Wqkv[2 * EMB:], bqkv[2 * EMB:].reshape(1, -1), Wo,
        bo.reshape(1, -1), Tw1, Tb1.reshape(1, -1), Tw2, Tb2.reshape(1, -1),
        pad8(Tw3), Tb3.reshape(1, -1), Iw1, Ib1.reshape(1, -1), Iw2,
        Ib2.reshape(1, -1), pad8(Iw3), Ib3.reshape(1, -1), Aw1,
        Ab1.reshape(1, -1),
        pad8(Aw2), jnp.concatenate([Ab2, jnp.zeros((6,), f32)]).reshape(1, -1))
    return (temp, iaw, anom, hf)
